# baseline (speedup 1.0000x reference)
import functools

import jax
import jax.numpy as jnp
from jax import lax
from jax.experimental import pallas as pl
from jax.experimental.pallas import tpu as pltpu
from jax.experimental.pallas import tpu_sc as plsc

NUM_FIELDS = 26
FIELD_DIM = 38462
EMBED_DIM = 16
BATCH = 16384
TOTAL = NUM_FIELDS * BATCH
VOCAB = NUM_FIELDS * FIELD_DIM
VPAD = 1000016
NC, NS = 2, 16
NW = NC * NS
ROWS_PER_STREAM = 128
NBLK = TOTAL // ROWS_PER_STREAM
NHALF = 2
NBLK_H = NBLK // NHALF
BLK_PER_W = NBLK_H // NW
CHUNK_BLKS = 4
NCHUNK = BLK_PER_W // CHUNK_BLKS


TCOLS = 768
NTCH = VOCAB // TCOLS
XTAIL_START = NTCH * TCOLS
XTAIL = VOCAB - XTAIL_START
NFULL_IT = NTCH // NW
NTBUF = 4


NWORDS = EMBED_DIM // 2


def _sc_transpose(embT, xtail):
    mesh = plsc.VectorSubcoreMesh(core_axis_name="c", subcore_axis_name="s")

    @functools.partial(
        pl.kernel,
        out_type=jax.ShapeDtypeStruct((VPAD * NWORDS,), jnp.int32),
        mesh=mesh,
        scratch_types=(
            [pltpu.VMEM((EMBED_DIM, TCOLS), jnp.float32)] * NTBUF
            + [pltpu.VMEM((TCOLS * NWORDS,), jnp.int32)] * NTBUF
            + [pltpu.VMEM((XTAIL * NWORDS,), jnp.int32)]
            + [pltpu.SemaphoreType.DMA] * (2 * NTBUF)
        ),
        compiler_params=pltpu.CompilerParams(needs_layout_passes=False),
    )
    def k(src_hbm, xtail_hbm, out_hbm, *rest):
        src_v = rest[0:NTBUF]
        dst_v = rest[NTBUF:2 * NTBUF]
        xt_v = rest[2 * NTBUF]
        sem_in = rest[2 * NTBUF + 1:2 * NTBUF + 1 + NTBUF]
        sem_out = rest[2 * NTBUF + 1 + NTBUF:]
        wid = lax.axis_index("s") * NC + lax.axis_index("c")
        lane = lax.iota(jnp.int32, 16) * NWORDS

        def transpose_buf(b, ncolgrp):
            def grp(j, carry):
                for w in range(NWORDS):
                    v0 = src_v[b][2 * w, pl.ds(j * 16, 16)]
                    v1 = src_v[b][2 * w + 1, pl.ds(j * 16, 16)]
                    pk = plsc.bitcast(
                        plsc.pack(v0, v1, format=plsc.PackFormat.INTERLEAVED),
                        jnp.int32)
                    plsc.store_scatter(dst_v[b],
                                       [lane + (j * 16 * NWORDS + w)], pk)
                return carry

            lax.fori_loop(0, ncolgrp, grp, 0)

        def start_in(i):
            col0 = (i * NW + wid) * TCOLS
            return pltpu.async_copy(src_hbm.at[:, pl.ds(col0, TCOLS)],
                                    src_v[i % NTBUF], sem_in[i % NTBUF])

        cp_in = [None] * NTBUF
        cp_out = [None] * NTBUF
        for i in range(NTBUF - 1):
            cp_in[i] = start_in(i)
        for i in range(NFULL_IT):
            b = i % NTBUF
            if i + NTBUF - 1 < NFULL_IT:
                cp_in[(i + NTBUF - 1) % NTBUF] = start_in(i + NTBUF - 1)
            cp_in[b].wait()
            if cp_out[b] is not None:
                cp_out[b].wait()
            transpose_buf(b, TCOLS // 16)
            col0 = (i * NW + wid) * TCOLS
            cp_out[b] = pltpu.async_copy(
                dst_v[b],
                out_hbm.at[pl.ds(col0 * NWORDS, TCOLS * NWORDS)],
                sem_out[b])
        for cp in cp_out:
            if cp is not None:
                cp.wait()

        @pl.when(wid < NTCH - NFULL_IT * NW)
        def _():
            col0 = (NFULL_IT * NW + wid) * TCOLS
            pltpu.sync_copy(src_hbm.at[:, pl.ds(col0, TCOLS)], src_v[0])
            transpose_buf(0, TCOLS // 16)
            pltpu.sync_copy(dst_v[0],
                            out_hbm.at[pl.ds(col0 * NWORDS,
                                             TCOLS * NWORDS)])

        @pl.when(wid == NTCH - NFULL_IT * NW)
        def _():
            pltpu.sync_copy(xtail_hbm, xt_v)
            pltpu.sync_copy(xt_v, out_hbm.at[pl.ds(XTAIL_START * NWORDS,
                                                   XTAIL * NWORDS)])

    return k(embT, xtail)


def _sc_gather(idx2, emb_lin, lin_flat):
    mesh = plsc.VectorSubcoreMesh(core_axis_name="c", subcore_axis_name="s")

    @functools.partial(
        pl.kernel,
        out_type=(
            jax.ShapeDtypeStruct((NBLK_H, ROWS_PER_STREAM, NWORDS),
                                 jnp.int32),
            jax.ShapeDtypeStruct((NBLK_H, ROWS_PER_STREAM), jnp.float32),
        ),
        mesh=mesh,
        scratch_types=[
            pltpu.VMEM((CHUNK_BLKS, ROWS_PER_STREAM), jnp.int32),
            pltpu.VMEM((CHUNK_BLKS, ROWS_PER_STREAM), jnp.int32),
            pltpu.VMEM((CHUNK_BLKS, ROWS_PER_STREAM, NWORDS), jnp.int32),
            pltpu.VMEM((CHUNK_BLKS, ROWS_PER_STREAM, NWORDS), jnp.int32),
            pltpu.VMEM((CHUNK_BLKS, ROWS_PER_STREAM), jnp.float32),
            pltpu.VMEM((CHUNK_BLKS, ROWS_PER_STREAM), jnp.float32),
            pltpu.SemaphoreType.DMA,
            pltpu.SemaphoreType.DMA,
            pltpu.SemaphoreType.DMA,
            pltpu.SemaphoreType.DMA,
            pltpu.SemaphoreType.DMA,
            pltpu.SemaphoreType.DMA,
            pltpu.SemaphoreType.DMA,
            pltpu.SemaphoreType.DMA,
        ],
        compiler_params=pltpu.CompilerParams(use_tc_tiling_on_sc=False),
    )
    def k(idx_hbm, emb_hbm, lin_hbm, emb_out, lin_out,
          idx_v0, idx_v1, rows_v0, rows_v1, lin_v0, lin_v1,
          sem_ge0, sem_ge1, sem_gl0, sem_gl1,
          sem_we0, sem_we1, sem_wl0, sem_wl1):
        wid = lax.axis_index("s") * NC + lax.axis_index("c")
        base = wid * BLK_PER_W
        idx_v = (idx_v0, idx_v1)
        rows_v = (rows_v0, rows_v1)
        lin_v = (lin_v0, lin_v1)
        sem_ge = (sem_ge0, sem_ge1)
        sem_gl = (sem_gl0, sem_gl1)
        sem_we = (sem_we0, sem_we1)
        sem_wl = (sem_wl0, sem_wl1)

        def start_chunk(c):
            b = c % 2
            off = base + c * CHUNK_BLKS
            pltpu.sync_copy(idx_hbm.at[pl.ds(off, CHUNK_BLKS)], idx_v[b])
            ge = [pltpu.async_copy(emb_hbm.at[idx_v[b].at[j]],
                                   rows_v[b].at[j], sem_ge[b])
                  for j in range(CHUNK_BLKS)]
            gl = [pltpu.async_copy(lin_hbm.at[idx_v[b].at[j]],
                                   lin_v[b].at[j], sem_gl[b])
                  for j in range(CHUNK_BLKS)]
            return ge + gl

        cp_g = [None, None]
        cp_w = [None, None]
        cp_g[0] = start_chunk(0)
        for c in range(NCHUNK):
            b = c % 2
            if c + 1 < NCHUNK:
                if cp_w[1 - b] is not None:
                    for w in cp_w[1 - b]:
                        w.wait()
                cp_g[1 - b] = start_chunk(c + 1)
            for g in cp_g[b]:
                g.wait()
            off = base + c * CHUNK_BLKS
            we = pltpu.async_copy(rows_v[b], emb_out.at[pl.ds(off, CHUNK_BLKS)],
                                  sem_we[b])
            wl = pltpu.async_copy(lin_v[b], lin_out.at[pl.ds(off, CHUNK_BLKS)],
                                  sem_wl[b])
            cp_w[b] = [we, wl]
        for ws in cp_w:
            if ws is not None:
                for w in ws:
                    w.wait()

    return k(idx2, emb_lin, lin_flat)


BB = 1024
MB = BB // 8
GRID = BATCH // BB


def _tc_body(embp_ref, linp_ref, bd1_ref, b1t_ref, bd2_ref, b2t_ref, bd3_ref,
             ones_ref, cb_ref, out_ref):
    eb16 = embp_ref[...]
    e = eb16.astype(jnp.float32)
    s = jnp.sum(e, axis=0)
    s2 = jnp.sum(e * e, axis=0)
    g = s * s - s2
    fm = 0.5 * jnp.dot(g.astype(jnp.bfloat16), ones_ref[...],
                       preferred_element_type=jnp.float32)
    eb = eb16.reshape(NUM_FIELDS * MB, 128)
    h = jnp.dot(eb, bd1_ref[...], preferred_element_type=jnp.float32)
    h = jnp.maximum(h + b1t_ref[...], 0.0)
    h2 = jnp.dot(h.astype(jnp.bfloat16), bd2_ref[...],
                 preferred_element_type=jnp.float32)
    h2 = jnp.maximum(h2 + b2t_ref[...], 0.0)
    h2s = jnp.sum(h2.reshape(NUM_FIELDS, MB, 256), axis=0)
    mlp = jnp.dot(h2s.astype(jnp.bfloat16), bd3_ref[...],
                  preferred_element_type=jnp.float32)
    lin = jnp.sum(linp_ref[...], axis=0)
    logits = lin + fm + mlp + cb_ref[0, 0]
    out_ref[...] = jax.nn.sigmoid(logits)


def _tc_dense(embp, linp, bd1, b1t, bd2, b2t, bd3, onesbd, cb):
    grid = embp.shape[1] // MB
    return pl.pallas_call(
        _tc_body,
        grid=(grid,),
        in_specs=[
            pl.BlockSpec((NUM_FIELDS, MB, 128), lambda i: (0, i, 0)),
            pl.BlockSpec((NUM_FIELDS, MB, 8), lambda i: (0, i, 0)),
            pl.BlockSpec((128, 512), lambda i: (0, 0)),
            pl.BlockSpec((1, 512), lambda i: (0, 0)),
            pl.BlockSpec((512, 256), lambda i: (0, 0)),
            pl.BlockSpec((1, 256), lambda i: (0, 0)),
            pl.BlockSpec((256, 8), lambda i: (0, 0)),
            pl.BlockSpec((128, 8), lambda i: (0, 0)),
            pl.BlockSpec((1, 1), lambda i: (0, 0)),
        ],
        out_specs=pl.BlockSpec((MB, 8), lambda i: (i, 0)),
        out_shape=jax.ShapeDtypeStruct((embp.shape[1], 8), jnp.float32),
    )(embp, linp, bd1, b1t, bd2, b2t, bd3, onesbd, cb)


def kernel(x, emb_table, lin_table, lin_bias, W1, b1, W2, b2, W3, b3):
    offsets = FIELD_DIM * jnp.arange(NUM_FIELDS, dtype=jnp.int32)
    idx2d = x.astype(jnp.int32).T + offsets[:, None]
    lin_flat = lin_table.reshape(-1)

    xtail = lax.bitcast_convert_type(
        emb_table[XTAIL_START:VOCAB].astype(jnp.bfloat16)
        .reshape(XTAIL * NWORDS, 2), jnp.int32)
    emb_lin = _sc_transpose(emb_table.T, xtail).reshape(VPAD, NWORDS)

    eye8 = jnp.eye(8, dtype=jnp.float32)
    bd1 = jnp.kron(eye8, W1).astype(jnp.bfloat16)
    bd2 = jnp.kron(eye8, W2).astype(jnp.bfloat16)
    bd3 = jnp.kron(eye8, W3).astype(jnp.bfloat16)
    onesbd = jnp.kron(eye8, jnp.ones((EMBED_DIM, 1), jnp.float32)).astype(
        jnp.bfloat16)
    b1t = jnp.tile(b1, 8).reshape(1, 512)
    b2t = jnp.tile(b2, 8).reshape(1, 256)
    cb = (lin_bias[0] + NUM_FIELDS * b3[0]).reshape(1, 1)

    halves = []
    bs = BATCH // NHALF
    bh = bs // 8
    for h in range(NHALF):
        idx_h = idx2d[:, h * bs:(h + 1) * bs].reshape(NBLK_H, ROWS_PER_STREAM)
        emb_words, lin_rows = _sc_gather(idx_h, emb_lin, lin_flat)
        embp = lax.bitcast_convert_type(emb_words, jnp.bfloat16).reshape(
            NUM_FIELDS, bh, 128)
        linp = lin_rows.reshape(NUM_FIELDS, bh, 8)
        halves.append(_tc_dense(embp, linp, bd1, b1t, bd2, b2t, bd3,
                                onesbd, cb))
    return jnp.concatenate(halves, axis=0).reshape(BATCH)


# speedup vs baseline: 11.3469x; 11.3469x over previous
import functools

import jax
import jax.numpy as jnp
from jax import lax
from jax.experimental import pallas as pl
from jax.experimental.pallas import tpu as pltpu
from jax.experimental.pallas import tpu_sc as plsc

NUM_FIELDS = 26
FIELD_DIM = 38462
EMBED_DIM = 16
BATCH = 16384
TOTAL = NUM_FIELDS * BATCH
VOCAB = NUM_FIELDS * FIELD_DIM
VPAD = 1000016
NC, NS = 2, 16
NW = NC * NS
ROWS_PER_STREAM = 128
NBLK = TOTAL // ROWS_PER_STREAM
NHALF = 1
NBLK_H = NBLK // NHALF
BLK_PER_W = NBLK_H // NW
CHUNK_BLKS = 8
NCHUNK = BLK_PER_W // CHUNK_BLKS


TCOLS = 768
NTCH = VOCAB // TCOLS
XTAIL_START = NTCH * TCOLS
XTAIL = VOCAB - XTAIL_START
NFULL_IT = NTCH // NW
NTBUF = 4


def _sc_transpose(embT, xtail):
    mesh = plsc.VectorSubcoreMesh(core_axis_name="c", subcore_axis_name="s")

    @functools.partial(
        pl.kernel,
        out_type=jax.ShapeDtypeStruct((VPAD * EMBED_DIM,), jnp.float32),
        mesh=mesh,
        scratch_types=(
            [pltpu.VMEM((EMBED_DIM, TCOLS), jnp.float32)] * NTBUF
            + [pltpu.VMEM((TCOLS * EMBED_DIM,), jnp.float32)] * NTBUF
            + [pltpu.VMEM((XTAIL * EMBED_DIM,), jnp.float32)]
            + [pltpu.SemaphoreType.DMA] * (2 * NTBUF)
        ),
        compiler_params=pltpu.CompilerParams(needs_layout_passes=False),
    )
    def k(src_hbm, xtail_hbm, out_hbm, *rest):
        src_v = rest[0:NTBUF]
        dst_v = rest[NTBUF:2 * NTBUF]
        xt_v = rest[2 * NTBUF]
        sem_in = rest[2 * NTBUF + 1:2 * NTBUF + 1 + NTBUF]
        sem_out = rest[2 * NTBUF + 1 + NTBUF:]
        wid = lax.axis_index("s") * NC + lax.axis_index("c")
        lane = lax.iota(jnp.int32, 16) * EMBED_DIM

        def transpose_buf(b, ncolgrp):
            def grp(j, carry):
                for d in range(EMBED_DIM):
                    v = src_v[b][d, pl.ds(j * 16, 16)]
                    plsc.store_scatter(dst_v[b], [lane + (j * 256 + d)], v)
                return carry

            lax.fori_loop(0, ncolgrp, grp, 0)

        def start_in(i):
            col0 = (i * NW + wid) * TCOLS
            return pltpu.async_copy(src_hbm.at[:, pl.ds(col0, TCOLS)],
                                    src_v[i % NTBUF], sem_in[i % NTBUF])

        cp_in = [None] * NTBUF
        cp_out = [None] * NTBUF
        for i in range(NTBUF - 1):
            cp_in[i] = start_in(i)
        for i in range(NFULL_IT):
            b = i % NTBUF
            if i + NTBUF - 1 < NFULL_IT:
                cp_in[(i + NTBUF - 1) % NTBUF] = start_in(i + NTBUF - 1)
            cp_in[b].wait()
            if cp_out[b] is not None:
                cp_out[b].wait()
            transpose_buf(b, TCOLS // 16)
            col0 = (i * NW + wid) * TCOLS
            cp_out[b] = pltpu.async_copy(
                dst_v[b],
                out_hbm.at[pl.ds(col0 * EMBED_DIM, TCOLS * EMBED_DIM)],
                sem_out[b])
        for cp in cp_out:
            if cp is not None:
                cp.wait()

        @pl.when(wid < NTCH - NFULL_IT * NW)
        def _():
            col0 = (NFULL_IT * NW + wid) * TCOLS
            pltpu.sync_copy(src_hbm.at[:, pl.ds(col0, TCOLS)], src_v[0])
            transpose_buf(0, TCOLS // 16)
            pltpu.sync_copy(dst_v[0],
                            out_hbm.at[pl.ds(col0 * EMBED_DIM,
                                             TCOLS * EMBED_DIM)])

        @pl.when(wid == NTCH - NFULL_IT * NW)
        def _():
            pltpu.sync_copy(xtail_hbm, xt_v)
            pltpu.sync_copy(xt_v, out_hbm.at[pl.ds(XTAIL_START * EMBED_DIM,
                                                   XTAIL * EMBED_DIM)])

    return k(embT, xtail)


def _sc_gather(idx2, emb_lin, lin_flat):
    mesh = plsc.VectorSubcoreMesh(core_axis_name="c", subcore_axis_name="s")

    @functools.partial(
        pl.kernel,
        out_type=(
            jax.ShapeDtypeStruct((NBLK_H, ROWS_PER_STREAM, EMBED_DIM),
                                 jnp.float32),
            jax.ShapeDtypeStruct((NBLK_H, ROWS_PER_STREAM), jnp.float32),
        ),
        mesh=mesh,
        scratch_types=[
            pltpu.VMEM((CHUNK_BLKS, ROWS_PER_STREAM), jnp.int32),
            pltpu.VMEM((CHUNK_BLKS, ROWS_PER_STREAM), jnp.int32),
            pltpu.VMEM((CHUNK_BLKS, ROWS_PER_STREAM, EMBED_DIM), jnp.float32),
            pltpu.VMEM((CHUNK_BLKS, ROWS_PER_STREAM, EMBED_DIM), jnp.float32),
            pltpu.VMEM((CHUNK_BLKS, ROWS_PER_STREAM), jnp.float32),
            pltpu.VMEM((CHUNK_BLKS, ROWS_PER_STREAM), jnp.float32),
            pltpu.SemaphoreType.DMA,
            pltpu.SemaphoreType.DMA,
            pltpu.SemaphoreType.DMA,
            pltpu.SemaphoreType.DMA,
            pltpu.SemaphoreType.DMA,
            pltpu.SemaphoreType.DMA,
            pltpu.SemaphoreType.DMA,
            pltpu.SemaphoreType.DMA,
        ],
        compiler_params=pltpu.CompilerParams(use_tc_tiling_on_sc=False),
    )
    def k(idx_hbm, emb_hbm, lin_hbm, emb_out, lin_out,
          idx_v0, idx_v1, rows_v0, rows_v1, lin_v0, lin_v1,
          sem_ge0, sem_ge1, sem_gl0, sem_gl1,
          sem_we0, sem_we1, sem_wl0, sem_wl1):
        wid = lax.axis_index("s") * NC + lax.axis_index("c")
        base = wid * BLK_PER_W
        idx_v = (idx_v0, idx_v1)
        rows_v = (rows_v0, rows_v1)
        lin_v = (lin_v0, lin_v1)
        sem_ge = (sem_ge0, sem_ge1)
        sem_gl = (sem_gl0, sem_gl1)
        sem_we = (sem_we0, sem_we1)
        sem_wl = (sem_wl0, sem_wl1)

        def start_chunk(c):
            b = c % 2
            off = base + c * CHUNK_BLKS
            pltpu.sync_copy(idx_hbm.at[pl.ds(off, CHUNK_BLKS)], idx_v[b])
            ge = [pltpu.async_copy(emb_hbm.at[idx_v[b].at[j]],
                                   rows_v[b].at[j], sem_ge[b])
                  for j in range(CHUNK_BLKS)]
            gl = [pltpu.async_copy(lin_hbm.at[idx_v[b].at[j]],
                                   lin_v[b].at[j], sem_gl[b])
                  for j in range(CHUNK_BLKS)]
            return ge + gl

        cp_g = [None, None]
        cp_w = [None, None]
        cp_g[0] = start_chunk(0)
        for c in range(NCHUNK):
            b = c % 2
            if c + 1 < NCHUNK:
                if cp_w[1 - b] is not None:
                    for w in cp_w[1 - b]:
                        w.wait()
                cp_g[1 - b] = start_chunk(c + 1)
            for g in cp_g[b]:
                g.wait()
            off = base + c * CHUNK_BLKS
            we = pltpu.async_copy(rows_v[b], emb_out.at[pl.ds(off, CHUNK_BLKS)],
                                  sem_we[b])
            wl = pltpu.async_copy(lin_v[b], lin_out.at[pl.ds(off, CHUNK_BLKS)],
                                  sem_wl[b])
            cp_w[b] = [we, wl]
        for ws in cp_w:
            if ws is not None:
                for w in ws:
                    w.wait()

    return k(idx2, emb_lin, lin_flat)


BB = 1024
MB = BB // 8
GRID = BATCH // BB


def _tc_body(embp_ref, linp_ref, bd1_ref, b1t_ref, bd2_ref, b2t_ref, bd3_ref,
             ones_ref, cb_ref, out_ref):
    e = embp_ref[...]
    s = jnp.sum(e, axis=0)
    s2 = jnp.sum(e * e, axis=0)
    g = s * s - s2
    fm = 0.5 * jnp.dot(g.astype(jnp.bfloat16), ones_ref[...],
                       preferred_element_type=jnp.float32)
    eb = e.reshape(NUM_FIELDS * MB, 128).astype(jnp.bfloat16)
    h = jnp.dot(eb, bd1_ref[...], preferred_element_type=jnp.float32)
    h = jnp.maximum(h + b1t_ref[...], 0.0)
    h2 = jnp.dot(h.astype(jnp.bfloat16), bd2_ref[...],
                 preferred_element_type=jnp.float32)
    h2 = jnp.maximum(h2 + b2t_ref[...], 0.0)
    h2s = jnp.sum(h2.reshape(NUM_FIELDS, MB, 256), axis=0)
    mlp = jnp.dot(h2s.astype(jnp.bfloat16), bd3_ref[...],
                  preferred_element_type=jnp.float32)
    lin = jnp.sum(linp_ref[...], axis=0)
    logits = lin + fm + mlp + cb_ref[0, 0]
    out_ref[...] = jax.nn.sigmoid(logits)


def _tc_dense(embp, linp, bd1, b1t, bd2, b2t, bd3, onesbd, cb):
    grid = embp.shape[1] // MB
    return pl.pallas_call(
        _tc_body,
        grid=(grid,),
        in_specs=[
            pl.BlockSpec((NUM_FIELDS, MB, 128), lambda i: (0, i, 0)),
            pl.BlockSpec((NUM_FIELDS, MB, 8), lambda i: (0, i, 0)),
            pl.BlockSpec((128, 512), lambda i: (0, 0)),
            pl.BlockSpec((1, 512), lambda i: (0, 0)),
            pl.BlockSpec((512, 256), lambda i: (0, 0)),
            pl.BlockSpec((1, 256), lambda i: (0, 0)),
            pl.BlockSpec((256, 8), lambda i: (0, 0)),
            pl.BlockSpec((128, 8), lambda i: (0, 0)),
            pl.BlockSpec((1, 1), lambda i: (0, 0)),
        ],
        out_specs=pl.BlockSpec((MB, 8), lambda i: (i, 0)),
        out_shape=jax.ShapeDtypeStruct((embp.shape[1], 8), jnp.float32),
    )(embp, linp, bd1, b1t, bd2, b2t, bd3, onesbd, cb)


def kernel(x, emb_table, lin_table, lin_bias, W1, b1, W2, b2, W3, b3):
    offsets = FIELD_DIM * jnp.arange(NUM_FIELDS, dtype=jnp.int32)
    idx2d = x.astype(jnp.int32).T + offsets[:, None]
    lin_flat = lin_table.reshape(-1)

    xtail = emb_table[XTAIL_START:VOCAB].reshape(XTAIL * EMBED_DIM)
    emb_lin = _sc_transpose(emb_table.T, xtail).reshape(VPAD, EMBED_DIM)

    eye8 = jnp.eye(8, dtype=jnp.float32)
    bd1 = jnp.kron(eye8, W1).astype(jnp.bfloat16)
    bd2 = jnp.kron(eye8, W2).astype(jnp.bfloat16)
    bd3 = jnp.kron(eye8, W3).astype(jnp.bfloat16)
    onesbd = jnp.kron(eye8, jnp.ones((EMBED_DIM, 1), jnp.float32)).astype(
        jnp.bfloat16)
    b1t = jnp.tile(b1, 8).reshape(1, 512)
    b2t = jnp.tile(b2, 8).reshape(1, 256)
    cb = (lin_bias[0] + NUM_FIELDS * b3[0]).reshape(1, 1)

    halves = []
    bs = BATCH // NHALF
    bh = bs // 8
    for h in range(NHALF):
        idx_h = idx2d[:, h * bs:(h + 1) * bs].reshape(NBLK_H, ROWS_PER_STREAM)
        emb_rows, lin_rows = _sc_gather(idx_h, emb_lin, lin_flat)
        embp = emb_rows.reshape(NUM_FIELDS, bh, 128)
        linp = lin_rows.reshape(NUM_FIELDS, bh, 8)
        halves.append(_tc_dense(embp, linp, bd1, b1t, bd2, b2t, bd3,
                                onesbd, cb))
    return jnp.concatenate(halves, axis=0).reshape(BATCH)


# speedup vs baseline: 11.4511x; 1.0092x over previous
import functools

import jax
import jax.numpy as jnp
from jax import lax
from jax.experimental import pallas as pl
from jax.experimental.pallas import tpu as pltpu
from jax.experimental.pallas import tpu_sc as plsc

NUM_FIELDS = 26
FIELD_DIM = 38462
EMBED_DIM = 16
BATCH = 16384
TOTAL = NUM_FIELDS * BATCH
VOCAB = NUM_FIELDS * FIELD_DIM
VPAD = 1000016
NC, NS = 2, 16
NW = NC * NS
ROWS_PER_STREAM = 128
NBLK = TOTAL // ROWS_PER_STREAM
NHALF = 1
NBLK_H = NBLK // NHALF
BLK_PER_W = NBLK_H // NW
CHUNK_BLKS = 8
NCHUNK = BLK_PER_W // CHUNK_BLKS


TCOLS = 768
NTCH = VOCAB // TCOLS
XTAIL_START = NTCH * TCOLS
XTAIL = VOCAB - XTAIL_START
NFULL_IT = NTCH // NW
NTBUF = 4


def _sc_transpose(embT, xtail):
    mesh = plsc.VectorSubcoreMesh(core_axis_name="c", subcore_axis_name="s")

    @functools.partial(
        pl.kernel,
        out_type=jax.ShapeDtypeStruct((VPAD * EMBED_DIM,), jnp.float32),
        mesh=mesh,
        scratch_types=(
            [pltpu.VMEM((EMBED_DIM, TCOLS), jnp.float32)] * NTBUF
            + [pltpu.VMEM((TCOLS * EMBED_DIM,), jnp.float32)] * NTBUF
            + [pltpu.VMEM((XTAIL * EMBED_DIM,), jnp.float32)]
            + [pltpu.SemaphoreType.DMA] * (2 * NTBUF)
        ),
        compiler_params=pltpu.CompilerParams(needs_layout_passes=False),
    )
    def k(src_hbm, xtail_hbm, out_hbm, *rest):
        src_v = rest[0:NTBUF]
        dst_v = rest[NTBUF:2 * NTBUF]
        xt_v = rest[2 * NTBUF]
        sem_in = rest[2 * NTBUF + 1:2 * NTBUF + 1 + NTBUF]
        sem_out = rest[2 * NTBUF + 1 + NTBUF:]
        wid = lax.axis_index("s") * NC + lax.axis_index("c")
        lane = lax.iota(jnp.int32, 16) * EMBED_DIM

        def transpose_buf(b, ncolgrp):
            def grp(j, carry):
                for d in range(EMBED_DIM):
                    v = src_v[b][d, pl.ds(j * 16, 16)]
                    plsc.store_scatter(dst_v[b], [lane + (j * 256 + d)], v)
                return carry

            lax.fori_loop(0, ncolgrp, grp, 0)

        def start_in(i):
            col0 = (i * NW + wid) * TCOLS
            return pltpu.async_copy(src_hbm.at[:, pl.ds(col0, TCOLS)],
                                    src_v[i % NTBUF], sem_in[i % NTBUF])

        cp_in = [None] * NTBUF
        cp_out = [None] * NTBUF
        for i in range(NTBUF - 1):
            cp_in[i] = start_in(i)
        for i in range(NFULL_IT):
            b = i % NTBUF
            if i + NTBUF - 1 < NFULL_IT:
                cp_in[(i + NTBUF - 1) % NTBUF] = start_in(i + NTBUF - 1)
            cp_in[b].wait()
            if cp_out[b] is not None:
                cp_out[b].wait()
            transpose_buf(b, TCOLS // 16)
            col0 = (i * NW + wid) * TCOLS
            cp_out[b] = pltpu.async_copy(
                dst_v[b],
                out_hbm.at[pl.ds(col0 * EMBED_DIM, TCOLS * EMBED_DIM)],
                sem_out[b])
        for cp in cp_out:
            if cp is not None:
                cp.wait()

        @pl.when(wid < NTCH - NFULL_IT * NW)
        def _():
            col0 = (NFULL_IT * NW + wid) * TCOLS
            pltpu.sync_copy(src_hbm.at[:, pl.ds(col0, TCOLS)], src_v[0])
            transpose_buf(0, TCOLS // 16)
            pltpu.sync_copy(dst_v[0],
                            out_hbm.at[pl.ds(col0 * EMBED_DIM,
                                             TCOLS * EMBED_DIM)])

        @pl.when(wid == NTCH - NFULL_IT * NW)
        def _():
            pltpu.sync_copy(xtail_hbm, xt_v)
            pltpu.sync_copy(xt_v, out_hbm.at[pl.ds(XTAIL_START * EMBED_DIM,
                                                   XTAIL * EMBED_DIM)])

    return k(embT, xtail)


def _sc_gather(idx2, emb_lin, lin_flat):
    mesh = plsc.VectorSubcoreMesh(core_axis_name="c", subcore_axis_name="s")

    @functools.partial(
        pl.kernel,
        out_type=(
            jax.ShapeDtypeStruct((NBLK_H, ROWS_PER_STREAM, EMBED_DIM),
                                 jnp.float32),
            jax.ShapeDtypeStruct((NBLK_H, ROWS_PER_STREAM), jnp.float32),
        ),
        mesh=mesh,
        scratch_types=[
            pltpu.VMEM((CHUNK_BLKS, ROWS_PER_STREAM), jnp.int32),
            pltpu.VMEM((CHUNK_BLKS, ROWS_PER_STREAM), jnp.int32),
            pltpu.VMEM((CHUNK_BLKS, ROWS_PER_STREAM, EMBED_DIM), jnp.float32),
            pltpu.VMEM((CHUNK_BLKS, ROWS_PER_STREAM, EMBED_DIM), jnp.float32),
            pltpu.VMEM((CHUNK_BLKS, ROWS_PER_STREAM), jnp.float32),
            pltpu.VMEM((CHUNK_BLKS, ROWS_PER_STREAM), jnp.float32),
            pltpu.VMEM_SHARED((VOCAB,), jnp.float32),
            pltpu.SemaphoreType.DMA,
            pltpu.SemaphoreType.DMA,
            pltpu.SemaphoreType.DMA,
            pltpu.SemaphoreType.DMA,
            pltpu.SemaphoreType.DMA,
            pltpu.SemaphoreType.DMA,
            pltpu.SemaphoreType.DMA,
            pltpu.SemaphoreType.DMA,
        ],
        compiler_params=pltpu.CompilerParams(use_tc_tiling_on_sc=False),
    )
    def k(idx_hbm, emb_hbm, lin_hbm, emb_out, lin_out,
          idx_v0, idx_v1, rows_v0, rows_v1, lin_v0, lin_v1, lin_sh,
          sem_ge0, sem_ge1, sem_gl0, sem_gl1,
          sem_we0, sem_we1, sem_wl0, sem_wl1):
        wid = lax.axis_index("s") * NC + lax.axis_index("c")
        base = wid * BLK_PER_W

        @pl.when(lax.axis_index("s") == 0)
        def _():
            pltpu.sync_copy(lin_hbm, lin_sh)

        plsc.subcore_barrier()
        idx_v = (idx_v0, idx_v1)
        rows_v = (rows_v0, rows_v1)
        lin_v = (lin_v0, lin_v1)
        sem_ge = (sem_ge0, sem_ge1)
        sem_gl = (sem_gl0, sem_gl1)
        sem_we = (sem_we0, sem_we1)
        sem_wl = (sem_wl0, sem_wl1)

        def start_chunk(c):
            b = c % 2
            off = base + c * CHUNK_BLKS
            pltpu.sync_copy(idx_hbm.at[pl.ds(off, CHUNK_BLKS)], idx_v[b])
            ge = [pltpu.async_copy(emb_hbm.at[idx_v[b].at[j]],
                                   rows_v[b].at[j], sem_ge[b])
                  for j in range(CHUNK_BLKS)]
            gl = [pltpu.async_copy(lin_sh.at[idx_v[b].at[j]],
                                   lin_v[b].at[j], sem_gl[b])
                  for j in range(CHUNK_BLKS)]
            return ge + gl

        cp_g = [None, None]
        cp_w = [None, None]
        cp_g[0] = start_chunk(0)
        for c in range(NCHUNK):
            b = c % 2
            if c + 1 < NCHUNK:
                if cp_w[1 - b] is not None:
                    for w in cp_w[1 - b]:
                        w.wait()
                cp_g[1 - b] = start_chunk(c + 1)
            for g in cp_g[b]:
                g.wait()
            off = base + c * CHUNK_BLKS
            we = pltpu.async_copy(rows_v[b], emb_out.at[pl.ds(off, CHUNK_BLKS)],
                                  sem_we[b])
            wl = pltpu.async_copy(lin_v[b], lin_out.at[pl.ds(off, CHUNK_BLKS)],
                                  sem_wl[b])
            cp_w[b] = [we, wl]
        for ws in cp_w:
            if ws is not None:
                for w in ws:
                    w.wait()

    return k(idx2, emb_lin, lin_flat)


BB = 1024
MB = BB // 8
GRID = BATCH // BB


def _tc_body(embp_ref, linp_ref, bd1_ref, b1t_ref, bd2_ref, b2t_ref, bd3_ref,
             ones_ref, cb_ref, out_ref):
    e = embp_ref[...]
    s = jnp.sum(e, axis=0)
    s2 = jnp.sum(e * e, axis=0)
    g = s * s - s2
    fm = 0.5 * jnp.dot(g.astype(jnp.bfloat16), ones_ref[...],
                       preferred_element_type=jnp.float32)
    eb = e.reshape(NUM_FIELDS * MB, 128).astype(jnp.bfloat16)
    h = jnp.dot(eb, bd1_ref[...], preferred_element_type=jnp.float32)
    h = jnp.maximum(h + b1t_ref[...], 0.0)
    h2 = jnp.dot(h.astype(jnp.bfloat16), bd2_ref[...],
                 preferred_element_type=jnp.float32)
    h2 = jnp.maximum(h2 + b2t_ref[...], 0.0)
    h2s = jnp.sum(h2.reshape(NUM_FIELDS, MB, 256), axis=0)
    mlp = jnp.dot(h2s.astype(jnp.bfloat16), bd3_ref[...],
                  preferred_element_type=jnp.float32)
    lin = jnp.sum(linp_ref[...], axis=0)
    logits = lin + fm + mlp + cb_ref[0, 0]
    out_ref[...] = jax.nn.sigmoid(logits)


def _tc_dense(embp, linp, bd1, b1t, bd2, b2t, bd3, onesbd, cb):
    grid = embp.shape[1] // MB
    return pl.pallas_call(
        _tc_body,
        grid=(grid,),
        in_specs=[
            pl.BlockSpec((NUM_FIELDS, MB, 128), lambda i: (0, i, 0)),
            pl.BlockSpec((NUM_FIELDS, MB, 8), lambda i: (0, i, 0)),
            pl.BlockSpec((128, 512), lambda i: (0, 0)),
            pl.BlockSpec((1, 512), lambda i: (0, 0)),
            pl.BlockSpec((512, 256), lambda i: (0, 0)),
            pl.BlockSpec((1, 256), lambda i: (0, 0)),
            pl.BlockSpec((256, 8), lambda i: (0, 0)),
            pl.BlockSpec((128, 8), lambda i: (0, 0)),
            pl.BlockSpec((1, 1), lambda i: (0, 0)),
        ],
        out_specs=pl.BlockSpec((MB, 8), lambda i: (i, 0)),
        out_shape=jax.ShapeDtypeStruct((embp.shape[1], 8), jnp.float32),
    )(embp, linp, bd1, b1t, bd2, b2t, bd3, onesbd, cb)


def kernel(x, emb_table, lin_table, lin_bias, W1, b1, W2, b2, W3, b3):
    offsets = FIELD_DIM * jnp.arange(NUM_FIELDS, dtype=jnp.int32)
    idx2d = x.astype(jnp.int32).T + offsets[:, None]
    lin_flat = lin_table.reshape(-1)

    xtail = emb_table[XTAIL_START:VOCAB].reshape(XTAIL * EMBED_DIM)
    emb_lin = _sc_transpose(emb_table.T, xtail).reshape(VPAD, EMBED_DIM)

    eye8 = jnp.eye(8, dtype=jnp.float32)
    bd1 = jnp.kron(eye8, W1).astype(jnp.bfloat16)
    bd2 = jnp.kron(eye8, W2).astype(jnp.bfloat16)
    bd3 = jnp.kron(eye8, W3).astype(jnp.bfloat16)
    onesbd = jnp.kron(eye8, jnp.ones((EMBED_DIM, 1), jnp.float32)).astype(
        jnp.bfloat16)
    b1t = jnp.tile(b1, 8).reshape(1, 512)
    b2t = jnp.tile(b2, 8).reshape(1, 256)
    cb = (lin_bias[0] + NUM_FIELDS * b3[0]).reshape(1, 1)

    halves = []
    bs = BATCH // NHALF
    bh = bs // 8
    for h in range(NHALF):
        idx_h = idx2d[:, h * bs:(h + 1) * bs].reshape(NBLK_H, ROWS_PER_STREAM)
        emb_rows, lin_rows = _sc_gather(idx_h, emb_lin, lin_flat)
        embp = emb_rows.reshape(NUM_FIELDS, bh, 128)
        linp = lin_rows.reshape(NUM_FIELDS, bh, 8)
        halves.append(_tc_dense(embp, linp, bd1, b1t, bd2, b2t, bd3,
                                onesbd, cb))
    return jnp.concatenate(halves, axis=0).reshape(BATCH)


# speedup vs baseline: 11.4744x; 1.0020x over previous
import functools

import jax
import jax.numpy as jnp
from jax import lax
from jax.experimental import pallas as pl
from jax.experimental.pallas import tpu as pltpu
from jax.experimental.pallas import tpu_sc as plsc

NUM_FIELDS = 26
FIELD_DIM = 38462
EMBED_DIM = 16
BATCH = 16384
TOTAL = NUM_FIELDS * BATCH
VOCAB = NUM_FIELDS * FIELD_DIM
VPAD = 1000016
NC, NS = 2, 16
NW = NC * NS
ROWS_PER_STREAM = 128
NBLK = TOTAL // ROWS_PER_STREAM
NHALF = 1
NBLK_H = NBLK // NHALF
BLK_PER_W = NBLK_H // NW
CHUNK_BLKS = 8
NCHUNK = BLK_PER_W // CHUNK_BLKS


TCOLS = 768
NTCH = VOCAB // TCOLS
XTAIL_START = NTCH * TCOLS
XTAIL = VOCAB - XTAIL_START
NFULL_IT = NTCH // NW
NTBUF = 4


def _sc_transpose(embT, xtail):
    mesh = plsc.VectorSubcoreMesh(core_axis_name="c", subcore_axis_name="s")

    @functools.partial(
        pl.kernel,
        out_type=jax.ShapeDtypeStruct((VPAD * EMBED_DIM,), jnp.float32),
        mesh=mesh,
        scratch_types=(
            [pltpu.VMEM((EMBED_DIM, TCOLS), jnp.float32)] * NTBUF
            + [pltpu.VMEM((TCOLS * EMBED_DIM,), jnp.float32)] * NTBUF
            + [pltpu.VMEM((XTAIL * EMBED_DIM,), jnp.float32)]
            + [pltpu.SemaphoreType.DMA] * (2 * NTBUF)
        ),
        compiler_params=pltpu.CompilerParams(needs_layout_passes=False),
    )
    def k(src_hbm, xtail_hbm, out_hbm, *rest):
        src_v = rest[0:NTBUF]
        dst_v = rest[NTBUF:2 * NTBUF]
        xt_v = rest[2 * NTBUF]
        sem_in = rest[2 * NTBUF + 1:2 * NTBUF + 1 + NTBUF]
        sem_out = rest[2 * NTBUF + 1 + NTBUF:]
        wid = lax.axis_index("s") * NC + lax.axis_index("c")
        lane = lax.iota(jnp.int32, 16) * EMBED_DIM

        def transpose_buf(b, ncolgrp):
            def grp(j, carry):
                for d in range(EMBED_DIM):
                    v = src_v[b][d, pl.ds(j * 16, 16)]
                    plsc.store_scatter(dst_v[b], [lane + (j * 256 + d)], v)
                return carry

            lax.fori_loop(0, ncolgrp, grp, 0)

        def start_in(i):
            col0 = (i * NW + wid) * TCOLS
            return pltpu.async_copy(src_hbm.at[:, pl.ds(col0, TCOLS)],
                                    src_v[i % NTBUF], sem_in[i % NTBUF])

        cp_in = [None] * NTBUF
        cp_out = [None] * NTBUF
        for i in range(NTBUF - 1):
            cp_in[i] = start_in(i)
        for i in range(NFULL_IT):
            b = i % NTBUF
            if i + NTBUF - 1 < NFULL_IT:
                cp_in[(i + NTBUF - 1) % NTBUF] = start_in(i + NTBUF - 1)
            cp_in[b].wait()
            if cp_out[b] is not None:
                cp_out[b].wait()
            transpose_buf(b, TCOLS // 16)
            col0 = (i * NW + wid) * TCOLS
            cp_out[b] = pltpu.async_copy(
                dst_v[b],
                out_hbm.at[pl.ds(col0 * EMBED_DIM, TCOLS * EMBED_DIM)],
                sem_out[b])
        for cp in cp_out:
            if cp is not None:
                cp.wait()

        @pl.when(wid < NTCH - NFULL_IT * NW)
        def _():
            col0 = (NFULL_IT * NW + wid) * TCOLS
            pltpu.sync_copy(src_hbm.at[:, pl.ds(col0, TCOLS)], src_v[0])
            transpose_buf(0, TCOLS // 16)
            pltpu.sync_copy(dst_v[0],
                            out_hbm.at[pl.ds(col0 * EMBED_DIM,
                                             TCOLS * EMBED_DIM)])

        @pl.when(wid == NTCH - NFULL_IT * NW)
        def _():
            pltpu.sync_copy(xtail_hbm, xt_v)
            pltpu.sync_copy(xt_v, out_hbm.at[pl.ds(XTAIL_START * EMBED_DIM,
                                                   XTAIL * EMBED_DIM)])

    return k(embT, xtail)


def _sc_gather(idx2, emb_lin, lin_flat):
    mesh = plsc.VectorSubcoreMesh(core_axis_name="c", subcore_axis_name="s")

    @functools.partial(
        pl.kernel,
        out_type=(
            jax.ShapeDtypeStruct((NBLK_H, ROWS_PER_STREAM, EMBED_DIM),
                                 jnp.float32),
            jax.ShapeDtypeStruct((NBLK_H, ROWS_PER_STREAM), jnp.float32),
        ),
        mesh=mesh,
        scratch_types=[
            pltpu.VMEM((CHUNK_BLKS, ROWS_PER_STREAM), jnp.int32),
            pltpu.VMEM((CHUNK_BLKS, ROWS_PER_STREAM), jnp.int32),
            pltpu.VMEM((CHUNK_BLKS, ROWS_PER_STREAM, EMBED_DIM), jnp.float32),
            pltpu.VMEM((CHUNK_BLKS, ROWS_PER_STREAM, EMBED_DIM), jnp.float32),
            pltpu.VMEM((CHUNK_BLKS, ROWS_PER_STREAM), jnp.float32),
            pltpu.VMEM((CHUNK_BLKS, ROWS_PER_STREAM), jnp.float32),
            pltpu.VMEM_SHARED((VOCAB,), jnp.float32),
            pltpu.SemaphoreType.DMA,
            pltpu.SemaphoreType.DMA,
            pltpu.SemaphoreType.DMA,
            pltpu.SemaphoreType.DMA,
            pltpu.SemaphoreType.DMA,
            pltpu.SemaphoreType.DMA,
            pltpu.SemaphoreType.DMA,
            pltpu.SemaphoreType.DMA,
        ],
        compiler_params=pltpu.CompilerParams(use_tc_tiling_on_sc=False),
    )
    def k(idx_hbm, emb_hbm, lin_hbm, emb_out, lin_out,
          idx_v0, idx_v1, rows_v0, rows_v1, lin_v0, lin_v1, lin_sh,
          sem_ge0, sem_ge1, sem_gl0, sem_gl1,
          sem_we0, sem_we1, sem_wl0, sem_wl1):
        wid = lax.axis_index("s") * NC + lax.axis_index("c")
        base = wid * BLK_PER_W

        @pl.when(lax.axis_index("s") == 0)
        def _():
            pltpu.sync_copy(lin_hbm, lin_sh)

        plsc.subcore_barrier()
        idx_v = (idx_v0, idx_v1)
        rows_v = (rows_v0, rows_v1)
        lin_v = (lin_v0, lin_v1)
        sem_ge = (sem_ge0, sem_ge1)
        sem_gl = (sem_gl0, sem_gl1)
        sem_we = (sem_we0, sem_we1)
        sem_wl = (sem_wl0, sem_wl1)

        def start_chunk(c):
            b = c % 2
            off = base + c * CHUNK_BLKS
            pltpu.sync_copy(idx_hbm.at[pl.ds(off, CHUNK_BLKS)], idx_v[b])
            ge = [pltpu.async_copy(emb_hbm.at[idx_v[b].at[j]],
                                   rows_v[b].at[j], sem_ge[b])
                  for j in range(CHUNK_BLKS)]
            gl = [pltpu.async_copy(lin_sh.at[idx_v[b].at[j]],
                                   lin_v[b].at[j], sem_gl[b])
                  for j in range(CHUNK_BLKS)]
            return ge + gl

        cp_g = [None, None]
        cp_w = [None, None]
        cp_g[0] = start_chunk(0)
        for c in range(NCHUNK):
            b = c % 2
            if c + 1 < NCHUNK:
                if cp_w[1 - b] is not None:
                    for w in cp_w[1 - b]:
                        w.wait()
                cp_g[1 - b] = start_chunk(c + 1)
            for g in cp_g[b]:
                g.wait()
            off = base + c * CHUNK_BLKS
            we = pltpu.async_copy(rows_v[b], emb_out.at[pl.ds(off, CHUNK_BLKS)],
                                  sem_we[b])
            wl = pltpu.async_copy(lin_v[b], lin_out.at[pl.ds(off, CHUNK_BLKS)],
                                  sem_wl[b])
            cp_w[b] = [we, wl]
        for ws in cp_w:
            if ws is not None:
                for w in ws:
                    w.wait()

    return k(idx2, emb_lin, lin_flat)


BB = 2048
MB = BB // 8
GRID = BATCH // BB


def _tc_body(embp_ref, linp_ref, bd1_ref, b1t_ref, bd2_ref, b2t_ref, bd3_ref,
             ones_ref, cb_ref, out_ref):
    e = embp_ref[...]
    s = jnp.sum(e, axis=0)
    s2 = jnp.sum(e * e, axis=0)
    g = s * s - s2
    fm = 0.5 * jnp.dot(g.astype(jnp.bfloat16), ones_ref[...],
                       preferred_element_type=jnp.float32)
    eb = e.reshape(NUM_FIELDS * MB, 128).astype(jnp.bfloat16)
    h = jnp.dot(eb, bd1_ref[...], preferred_element_type=jnp.float32)
    h = jnp.maximum(h + b1t_ref[...], 0.0)
    h2 = jnp.dot(h.astype(jnp.bfloat16), bd2_ref[...],
                 preferred_element_type=jnp.float32)
    h2 = jnp.maximum(h2 + b2t_ref[...], 0.0)
    h2s = jnp.sum(h2.reshape(NUM_FIELDS, MB, 256), axis=0)
    mlp = jnp.dot(h2s.astype(jnp.bfloat16), bd3_ref[...],
                  preferred_element_type=jnp.float32)
    lin = jnp.sum(linp_ref[...], axis=0)
    logits = lin + fm + mlp + cb_ref[0, 0]
    out_ref[...] = jax.nn.sigmoid(logits)


def _tc_dense(embp, linp, bd1, b1t, bd2, b2t, bd3, onesbd, cb):
    grid = embp.shape[1] // MB
    return pl.pallas_call(
        _tc_body,
        grid=(grid,),
        in_specs=[
            pl.BlockSpec((NUM_FIELDS, MB, 128), lambda i: (0, i, 0)),
            pl.BlockSpec((NUM_FIELDS, MB, 8), lambda i: (0, i, 0)),
            pl.BlockSpec((128, 512), lambda i: (0, 0)),
            pl.BlockSpec((1, 512), lambda i: (0, 0)),
            pl.BlockSpec((512, 256), lambda i: (0, 0)),
            pl.BlockSpec((1, 256), lambda i: (0, 0)),
            pl.BlockSpec((256, 8), lambda i: (0, 0)),
            pl.BlockSpec((128, 8), lambda i: (0, 0)),
            pl.BlockSpec((1, 1), lambda i: (0, 0)),
        ],
        out_specs=pl.BlockSpec((MB, 8), lambda i: (i, 0)),
        out_shape=jax.ShapeDtypeStruct((embp.shape[1], 8), jnp.float32),
    )(embp, linp, bd1, b1t, bd2, b2t, bd3, onesbd, cb)


def kernel(x, emb_table, lin_table, lin_bias, W1, b1, W2, b2, W3, b3):
    offsets = FIELD_DIM * jnp.arange(NUM_FIELDS, dtype=jnp.int32)
    idx2d = x.astype(jnp.int32).T + offsets[:, None]
    lin_flat = lin_table.reshape(-1)

    xtail = emb_table[XTAIL_START:VOCAB].reshape(XTAIL * EMBED_DIM)
    emb_lin = _sc_transpose(emb_table.T, xtail).reshape(VPAD, EMBED_DIM)

    eye8 = jnp.eye(8, dtype=jnp.float32)
    bd1 = jnp.kron(eye8, W1).astype(jnp.bfloat16)
    bd2 = jnp.kron(eye8, W2).astype(jnp.bfloat16)
    bd3 = jnp.kron(eye8, W3).astype(jnp.bfloat16)
    onesbd = jnp.kron(eye8, jnp.ones((EMBED_DIM, 1), jnp.float32)).astype(
        jnp.bfloat16)
    b1t = jnp.tile(b1, 8).reshape(1, 512)
    b2t = jnp.tile(b2, 8).reshape(1, 256)
    cb = (lin_bias[0] + NUM_FIELDS * b3[0]).reshape(1, 1)

    halves = []
    bs = BATCH // NHALF
    bh = bs // 8
    for h in range(NHALF):
        idx_h = idx2d[:, h * bs:(h + 1) * bs].reshape(NBLK_H, ROWS_PER_STREAM)
        emb_rows, lin_rows = _sc_gather(idx_h, emb_lin, lin_flat)
        embp = emb_rows.reshape(NUM_FIELDS, bh, 128)
        linp = lin_rows.reshape(NUM_FIELDS, bh, 8)
        halves.append(_tc_dense(embp, linp, bd1, b1t, bd2, b2t, bd3,
                                onesbd, cb))
    return jnp.concatenate(halves, axis=0).reshape(BATCH)
